# Initial kernel scaffold; baseline (speedup 1.0000x reference)
import jax
import jax.numpy as jnp
from jax.experimental import pallas as pl


def kernel(x):
    raise NotImplementedError("write your pallas kernel here")


# speedup vs baseline: 9.6510x; 9.6510x over previous
import functools

import jax
import jax.numpy as jnp
from jax import lax
from jax.experimental import pallas as pl
from jax.experimental.pallas import tpu as pltpu
from jax.experimental.pallas import tpu_sc as plsc

BINS = 64
SMOOTHING = 0.1
CONFIDENCE = 1.0 - SMOOTHING
FILL = SMOOTHING / BINS

NC = 2
NS = 16
NW = NC * NS
LANES = 16

RNE_MAGIC = 12582912.0


def _sc_body(P, n_chunks_per_worker, chunks_per_batch, HW, x2_hbm, out_hbm,
             dbuf, sbuf, ibuf, tile, sem):
    wid = lax.axis_index("s") * NC + lax.axis_index("c")
    iota16 = lax.iota(jnp.int32, LANES)
    fill_vec = jnp.full((LANES,), FILL, jnp.float32)
    conf_vec = jnp.full((LANES,), CONFIDENCE, jnp.float32)
    n_grp = P // LANES

    def init_grp(i, _):
        tile[pl.ds(i * LANES, LANES)] = fill_vec
        return 0

    lax.fori_loop(0, BINS * n_grp, init_grp, 0)

    def chunk_body(t, _):
        g = wid * n_chunks_per_worker + t
        b = g // chunks_per_batch
        p0 = (g - b * chunks_per_batch) * P

        pltpu.sync_copy(x2_hbm.at[2 * b, pl.ds(p0, P)], dbuf)
        pltpu.sync_copy(x2_hbm.at[2 * b + 1, pl.ds(p0, P)], sbuf)

        def grp(i, _):
            off = i * LANES
            d = dbuf[pl.ds(off, LANES)]
            s = sbuf[pl.ds(off, LANES)]
            t2 = (0.5 * (d + 1.0)) * (BINS - 2.0)
            r = (t2 + RNE_MAGIC) - RNE_MAGIC
            idx = r.astype(jnp.int32) + 1
            idx = jnp.where(s > 0.0, idx, 0)
            ibuf[pl.ds(off, LANES)] = idx
            plsc.store_scatter(tile, [idx * P + (iota16 + off)], conf_vec)
            return 0

        lax.fori_loop(0, n_grp, grp, 0)

        out_base = b * (BINS * HW) + p0

        def fire(c, _):
            pltpu.async_copy(
                tile.at[pl.ds(c * P, P)],
                out_hbm.at[pl.ds(out_base + c * HW, P)],
                sem,
            )
            return 0

        lax.fori_loop(0, BINS, fire, 0)

        pltpu.make_async_copy(
            out_hbm.at[pl.ds(0, BINS * P)], tile, sem
        ).wait()

        def grp2(i, _):
            off = i * LANES
            idx = ibuf[pl.ds(off, LANES)]
            plsc.store_scatter(tile, [idx * P + (iota16 + off)], fill_vec)
            return 0

        lax.fori_loop(0, n_grp, grp2, 0)
        return 0

    lax.fori_loop(0, n_chunks_per_worker, chunk_body, 0)


def kernel(x):
    assert x.ndim >= 3 and x.shape[-3] == 2
    *other_dims, C, H, W = x.shape
    x = x.reshape((-1, C, H, W))
    B = x.shape[0]
    HW = H * W

    P = 1536
    assert HW % P == 0
    chunks_per_batch = HW // P
    total_chunks = B * chunks_per_batch
    assert total_chunks % NW == 0
    n_chunks_per_worker = total_chunks // NW

    x2 = x.reshape(B * C, HW)

    mesh = plsc.VectorSubcoreMesh(core_axis_name="c", subcore_axis_name="s")
    run = pl.kernel(
        functools.partial(_sc_body, P, n_chunks_per_worker, chunks_per_batch,
                          HW),
        out_type=jax.ShapeDtypeStruct((B * BINS * HW,), jnp.float32),
        mesh=mesh,
        scratch_types=[
            pltpu.VMEM((P,), jnp.float32),
            pltpu.VMEM((P,), jnp.float32),
            pltpu.VMEM((P,), jnp.int32),
            pltpu.VMEM((BINS * P,), jnp.float32),
            pltpu.SemaphoreType.DMA,
        ],
        compiler_params=pltpu.CompilerParams(needs_layout_passes=False),
    )
    out = run(x2)
    return out.reshape(tuple(other_dims) + (BINS, H, W))


# speedup vs baseline: 11.4849x; 1.1900x over previous
import functools

import jax
import jax.numpy as jnp
from jax import lax
from jax.experimental import pallas as pl
from jax.experimental.pallas import tpu as pltpu
from jax.experimental.pallas import tpu_sc as plsc

BINS = 64
SMOOTHING = 0.1
CONFIDENCE = 1.0 - SMOOTHING
FILL = SMOOTHING / BINS

NC = 2
NS = 16
NW = NC * NS
LANES = 16

RNE_MAGIC = 12582912.0


def _sc_body(P, n_chunks_per_worker, chunks_per_batch, x2_hbm, out_hbm,
             dbuf, sbuf, ibuf, tile, sem):
    wid = lax.axis_index("s") * NC + lax.axis_index("c")
    iota16 = lax.iota(jnp.int32, LANES)
    fill_vec = jnp.full((LANES,), FILL, jnp.float32)
    conf_vec = jnp.full((LANES,), CONFIDENCE, jnp.float32)
    n_grp = P // LANES

    def init_row(r, _):
        def init_grp(i, _):
            tile[r, pl.ds(i * LANES, LANES)] = fill_vec
            return 0

        return lax.fori_loop(0, n_grp, init_grp, 0)

    lax.fori_loop(0, BINS, init_row, 0)

    def chunk_body(t, _):
        g = wid * n_chunks_per_worker + t
        b = g // chunks_per_batch
        p0 = (g - b * chunks_per_batch) * P

        pltpu.sync_copy(x2_hbm.at[2 * b, pl.ds(p0, P)], dbuf)
        pltpu.sync_copy(x2_hbm.at[2 * b + 1, pl.ds(p0, P)], sbuf)

        def grp(i, _):
            off = i * LANES
            d = dbuf[pl.ds(off, LANES)]
            s = sbuf[pl.ds(off, LANES)]
            t2 = (0.5 * (d + 1.0)) * (BINS - 2.0)
            r = (t2 + RNE_MAGIC) - RNE_MAGIC
            idx = r.astype(jnp.int32) + 1
            idx = jnp.where(s > 0.0, idx, 0)
            ibuf[pl.ds(off, LANES)] = idx
            plsc.store_scatter(tile, [idx, iota16 + off], conf_vec)
            return 0

        lax.fori_loop(0, n_grp, grp, 0)

        pltpu.async_copy(tile, out_hbm.at[b, :, pl.ds(p0, P)], sem).wait()

        def grp2(i, _):
            off = i * LANES
            idx = ibuf[pl.ds(off, LANES)]
            plsc.store_scatter(tile, [idx, iota16 + off], fill_vec)
            return 0

        lax.fori_loop(0, n_grp, grp2, 0)
        return 0

    lax.fori_loop(0, n_chunks_per_worker, chunk_body, 0)


def kernel(x):
    assert x.ndim >= 3 and x.shape[-3] == 2
    *other_dims, C, H, W = x.shape
    x = x.reshape((-1, C, H, W))
    B = x.shape[0]
    HW = H * W

    P = 1536
    assert HW % P == 0
    chunks_per_batch = HW // P
    total_chunks = B * chunks_per_batch
    assert total_chunks % NW == 0
    n_chunks_per_worker = total_chunks // NW

    x2 = x.reshape(B * C, HW)

    mesh = plsc.VectorSubcoreMesh(core_axis_name="c", subcore_axis_name="s")
    run = pl.kernel(
        functools.partial(_sc_body, P, n_chunks_per_worker, chunks_per_batch),
        out_type=jax.ShapeDtypeStruct((B, BINS, HW), jnp.float32),
        mesh=mesh,
        scratch_types=[
            pltpu.VMEM((P,), jnp.float32),
            pltpu.VMEM((P,), jnp.float32),
            pltpu.VMEM((P,), jnp.int32),
            pltpu.VMEM((BINS, P), jnp.float32),
            pltpu.SemaphoreType.DMA,
        ],
        compiler_params=pltpu.CompilerParams(needs_layout_passes=False),
    )
    out = run(x2)
    return out.reshape(tuple(other_dims) + (BINS, H, W))


# speedup vs baseline: 14.2941x; 1.2446x over previous
import functools

import jax
import jax.numpy as jnp
from jax import lax
from jax.experimental import pallas as pl
from jax.experimental.pallas import tpu as pltpu
from jax.experimental.pallas import tpu_sc as plsc

BINS = 64
SMOOTHING = 0.1
CONFIDENCE = 1.0 - SMOOTHING
FILL = SMOOTHING / BINS

NC = 2
NS = 16
NW = NC * NS
LANES = 16

P = 768
SUPER = 6
PAIRS = SUPER // 2

RNE_MAGIC = 12582912.0


def _sc_body(HW, n_super, wpb, x2_hbm, out_hbm,
             dbig, sbig, ibuf0, ibuf1, tile0, tile1, sem0, sem1):
    wid = lax.axis_index("s") * NC + lax.axis_index("c")
    iota16 = lax.iota(jnp.int32, LANES)
    fill_vec = jnp.full((LANES,), FILL, jnp.float32)
    conf_vec = jnp.full((LANES,), CONFIDENCE, jnp.float32)
    n_grp = P // LANES
    worker_pixels = n_super * SUPER * P

    b = wid // wpb
    p_base = (wid - b * wpb) * worker_pixels

    def init_row(r, _):
        def init_grp(i, _):
            tile0[r, pl.ds(i * LANES, LANES)] = fill_vec
            tile1[r, pl.ds(i * LANES, LANES)] = fill_vec
            return 0

        return lax.fori_loop(0, n_grp, init_grp, 0)

    lax.fori_loop(0, BINS, init_row, 0)

    def process(tile, ibuf, sem, not_first, q0, p0):
        @pl.when(not_first)
        def _():
            pltpu.make_async_copy(
                tile, out_hbm.at[b, :, pl.ds(0, P)], sem
            ).wait()

            def grp2(i, _):
                off = i * LANES
                idx = ibuf[pl.ds(off, LANES)]
                plsc.store_scatter(tile, [idx, iota16 + off], fill_vec)
                return 0

            lax.fori_loop(0, n_grp, grp2, 0)

        def grp(i, _):
            off = i * LANES
            d = dbig[pl.ds(q0 + off, LANES)]
            s = sbig[pl.ds(q0 + off, LANES)]
            t2 = (0.5 * (d + 1.0)) * (BINS - 2.0)
            r = (t2 + RNE_MAGIC) - RNE_MAGIC
            idx = r.astype(jnp.int32) + 1
            idx = jnp.where(s > 0.0, idx, 0)
            ibuf[pl.ds(off, LANES)] = idx
            plsc.store_scatter(tile, [idx, iota16 + off], conf_vec)
            return 0

        lax.fori_loop(0, n_grp, grp, 0)
        pltpu.async_copy(tile, out_hbm.at[b, :, pl.ds(p0, P)], sem)

    def super_body(s, _):
        sp0 = p_base + s * (SUPER * P)
        pltpu.sync_copy(x2_hbm.at[2 * b, pl.ds(sp0, SUPER * P)], dbig)
        pltpu.sync_copy(x2_hbm.at[2 * b + 1, pl.ds(sp0, SUPER * P)], sbig)

        def pair_body(j, _):
            not_first = (s * PAIRS + j) > 0
            q0 = j * (2 * P)
            process(tile0, ibuf0, sem0, not_first, q0, sp0 + q0)
            process(tile1, ibuf1, sem1, not_first, q0 + P, sp0 + q0 + P)
            return 0

        return lax.fori_loop(0, PAIRS, pair_body, 0)

    lax.fori_loop(0, n_super, super_body, 0)

    pltpu.make_async_copy(tile0, out_hbm.at[b, :, pl.ds(0, P)], sem0).wait()
    pltpu.make_async_copy(tile1, out_hbm.at[b, :, pl.ds(0, P)], sem1).wait()


def kernel(x):
    assert x.ndim >= 3 and x.shape[-3] == 2
    *other_dims, C, H, W = x.shape
    x = x.reshape((-1, C, H, W))
    B = x.shape[0]
    HW = H * W

    assert NW % B == 0
    wpb = NW // B
    assert HW % wpb == 0
    worker_pixels = HW // wpb
    assert worker_pixels % (SUPER * P) == 0
    n_super = worker_pixels // (SUPER * P)

    x2 = x.reshape(B * C, HW)

    mesh = plsc.VectorSubcoreMesh(core_axis_name="c", subcore_axis_name="s")
    run = pl.kernel(
        functools.partial(_sc_body, HW, n_super, wpb),
        out_type=jax.ShapeDtypeStruct((B, BINS, HW), jnp.float32),
        mesh=mesh,
        scratch_types=[
            pltpu.VMEM((SUPER * P,), jnp.float32),
            pltpu.VMEM((SUPER * P,), jnp.float32),
            pltpu.VMEM((P,), jnp.int32),
            pltpu.VMEM((P,), jnp.int32),
            pltpu.VMEM((BINS, P), jnp.float32),
            pltpu.VMEM((BINS, P), jnp.float32),
            pltpu.SemaphoreType.DMA,
            pltpu.SemaphoreType.DMA,
        ],
        compiler_params=pltpu.CompilerParams(needs_layout_passes=False),
    )
    out = run(x2)
    return out.reshape(tuple(other_dims) + (BINS, H, W))
